# baseline (speedup 1.0000x reference)
import functools

import jax
import jax.numpy as jnp
from jax import lax
from jax.experimental import pallas as pl
from jax.experimental.pallas import tpu as pltpu
from jax.experimental.pallas import tpu_sc as plsc

_NC = 2
_NS = 16
_NW = _NC * _NS
_CH = 128
_NB = 5
_L = 16


def _lookup_call(n_chunks, V, H, D):
    c_per_w = n_chunks // _NW
    n_groups = c_per_w // _NB
    bt_n = _CH // _L
    dt_n = D // 8
    mesh = plsc.VectorSubcoreMesh(core_axis_name="c", subcore_axis_name="s")

    @functools.partial(
        pl.kernel,
        mesh=mesh,
        compiler_params=pltpu.CompilerParams(use_tc_tiling_on_sc=False,
                                             needs_layout_passes=False),
        out_type=jax.ShapeDtypeStruct((H, dt_n, n_chunks // H, 8, _CH),
                                      jnp.float32),
        scratch_types=[
            pltpu.VMEM((c_per_w * _CH,), jnp.int32),
            pltpu.VMEM((_NB, _CH, D), jnp.float32),
            pltpu.VMEM((_NB, dt_n, 8, _CH), jnp.float32),
            pltpu.SemaphoreType.DMA,
            pltpu.SemaphoreType.DMA,
        ],
    )
    def k(idx_hbm, tab_hbm, out_hbm, idx_v, rows_v, tile_v, gsem, ssem):
        wid = lax.axis_index("s") * _NC + lax.axis_index("c")
        base = wid * c_per_w
        n_bt = n_chunks // H
        pltpu.sync_copy(idx_hbm.at[pl.ds(base * _CH, c_per_w * _CH)], idx_v)

        def start_gather(j, b):
            idx_chunk = idx_v.at[pl.ds(j * _CH, _CH)]
            pltpu.make_async_copy(tab_hbm.at[idx_chunk], rows_v.at[b],
                                  gsem).start()

        def wait_gather(b):
            pltpu.make_async_copy(tab_hbm.at[idx_v.at[pl.ds(0, _CH)]],
                                  rows_v.at[b], gsem).wait()

        def start_store(j, b):
            c = base + j
            h = c // n_bt
            bt = lax.rem(c, n_bt)
            pltpu.make_async_copy(tile_v.at[b], out_hbm.at[h, :, bt],
                                  ssem).start()

        def wait_store(b):
            pltpu.make_async_copy(tile_v.at[b], out_hbm.at[0, :, 0],
                                  ssem).wait()

        rowvs = [jnp.arange(_L, dtype=jnp.int32) + (g * _L) for g in range(bt_n)]

        def transpose(b):
            src = rows_v.at[b]
            dst = tile_v.at[b]

            def dtbody(dt, carry):
                dbase = dt * 8
                for dl in range(8):
                    colv = jnp.full((_L,), dbase + dl, dtype=jnp.int32)
                    for g in range(bt_n):
                        val = plsc.load_gather(src, [rowvs[g], colv])
                        dst[dt, dl, pl.ds(g * _L, _L)] = val
                return carry

            lax.fori_loop(0, dt_n, dtbody, 0)

        for b in range(_NB):
            start_gather(b, b)

        def body(g, carry):
            for b in range(_NB):
                j = g * _NB + b
                wait_gather(b)

                @pl.when(g > 0)
                def _():
                    wait_store(b)

                transpose(b)
                start_store(j, b)

                @pl.when(g < n_groups - 1)
                def _():
                    start_gather(j + _NB, b)

            return carry

        lax.fori_loop(0, n_groups, body, 0)

        for b in range(_NB):
            wait_store(b)

    return k


def kernel(x, lookup_dict):
    B, H = x.shape
    V, D = lookup_dict.shape
    n = B * H
    n_chunks = n // _CH
    assert n % (_CH * _NW) == 0 and B % _CH == 0 and D % 8 == 0
    idx_flat = x.T.reshape(n).astype(jnp.int32)
    y = _lookup_call(n_chunks, V, H, D)(idx_flat, lookup_dict)
    return y.transpose(2, 4, 0, 1, 3).reshape(B, H, D)


# speedup vs baseline: 1.9644x; 1.9644x over previous
import functools

import jax
import jax.numpy as jnp
from jax import lax
from jax.experimental import pallas as pl
from jax.experimental.pallas import tpu as pltpu
from jax.experimental.pallas import tpu_sc as plsc

_NC = 2
_NS = 16
_NW = _NC * _NS
_CH = 128
_NB = 5
_L = 16


def _lookup_call(n_chunks, V, H, D):
    c_per_w = n_chunks // _NW
    n_groups = c_per_w // _NB
    bt_n = _CH // _L
    dt_n = D // 8
    mesh = plsc.VectorSubcoreMesh(core_axis_name="c", subcore_axis_name="s")

    @functools.partial(
        pl.kernel,
        mesh=mesh,
        compiler_params=pltpu.CompilerParams(use_tc_tiling_on_sc=False,
                                             needs_layout_passes=False),
        out_type=jax.ShapeDtypeStruct((H, dt_n, n_chunks // H, 8, _CH),
                                      jnp.float32),
        scratch_types=[
            pltpu.VMEM((c_per_w * _CH,), jnp.int32),
            pltpu.VMEM((_NB, _CH, D), jnp.float32),
            pltpu.VMEM((_NB, dt_n, 8, _CH), jnp.float32),
            pltpu.SemaphoreType.DMA,
            pltpu.SemaphoreType.DMA,
        ],
    )
    def k(idx_hbm, tab_hbm, out_hbm, idx_v, rows_v, tile_v, gsem, ssem):
        wid = lax.axis_index("s") * _NC + lax.axis_index("c")
        base = wid * c_per_w
        n_bt = n_chunks // H
        pltpu.sync_copy(idx_hbm.at[pl.ds(base * _CH, c_per_w * _CH)], idx_v)

        def start_gather(j, b):
            idx_chunk = idx_v.at[pl.ds(j * _CH, _CH)]
            pltpu.make_async_copy(tab_hbm.at[idx_chunk], rows_v.at[b],
                                  gsem).start()

        def wait_gather(b):
            pltpu.make_async_copy(tab_hbm.at[idx_v.at[pl.ds(0, _CH)]],
                                  rows_v.at[b], gsem).wait()

        def start_store(j, b):
            c = base + j
            h = c // n_bt
            bt = lax.rem(c, n_bt)
            pltpu.make_async_copy(tile_v.at[b], out_hbm.at[h, :, bt],
                                  ssem).start()

        def wait_store(b):
            pltpu.make_async_copy(tile_v.at[b], out_hbm.at[0, :, 0],
                                  ssem).wait()

        iota = jnp.arange(_L, dtype=jnp.int32)
        perms = [(iota + k) & 15 for k in range(_L)]
        rowraw = [iota + _L * g for g in range(bt_n)]

        def transpose(b):
            src = rows_v.at[b]
            dst = tile_v.at[b]

            def cbody(c, carry):
                d0v = jnp.full((_L,), c * _L, dtype=jnp.int32)
                for k in range(_L):
                    dv = perms[k] + d0v
                    dtv = dv // 8
                    dlv = dv & 7
                    for g in range(bt_n):
                        val = plsc.load_gather(src, [rowraw[g], dv])
                        plsc.store_scatter(dst, [dtv, dlv, rowraw[g]], val)
                return carry

            lax.fori_loop(0, D // _L, cbody, 0)

        for b in range(_NB):
            start_gather(b, b)

        def body(g, carry):
            for b in range(_NB):
                j = g * _NB + b
                wait_gather(b)

                @pl.when(g > 0)
                def _():
                    wait_store(b)

                transpose(b)
                start_store(j, b)

                @pl.when(g < n_groups - 1)
                def _():
                    start_gather(j + _NB, b)

            return carry

        lax.fori_loop(0, n_groups, body, 0)

        for b in range(_NB):
            wait_store(b)

    return k


def kernel(x, lookup_dict):
    B, H = x.shape
    V, D = lookup_dict.shape
    n = B * H
    n_chunks = n // _CH
    assert n % (_CH * _NW) == 0 and B % _CH == 0 and D % 8 == 0
    idx_flat = x.T.reshape(n).astype(jnp.int32)
    y = _lookup_call(n_chunks, V, H, D)(idx_flat, lookup_dict)
    return y.transpose(2, 4, 0, 1, 3).reshape(B, H, D)


# speedup vs baseline: 2.1652x; 1.1022x over previous
import functools

import jax
import jax.numpy as jnp
from jax import lax
from jax.experimental import pallas as pl
from jax.experimental.pallas import tpu as pltpu
from jax.experimental.pallas import tpu_sc as plsc

_NC = 2
_NS = 16
_NW = _NC * _NS
_CH = 128
_NB = 5
_L = 16


def _lookup_call(n_chunks, V, H, D):
    c_per_w = n_chunks // _NW
    n_groups = c_per_w // _NB
    bt_n = _CH // _L
    dt_n = D // 8
    mesh = plsc.VectorSubcoreMesh(core_axis_name="c", subcore_axis_name="s")

    @functools.partial(
        pl.kernel,
        mesh=mesh,
        compiler_params=pltpu.CompilerParams(use_tc_tiling_on_sc=False,
                                             needs_layout_passes=False),
        out_type=jax.ShapeDtypeStruct((H, dt_n, n_chunks // H, 8, _CH),
                                      jnp.float32),
        scratch_types=[
            pltpu.VMEM((c_per_w * _CH,), jnp.int32),
            pltpu.VMEM((_NB, _CH, D), jnp.float32),
            pltpu.VMEM((_NB, D, _CH), jnp.float32),
            pltpu.SemaphoreType.DMA,
            pltpu.SemaphoreType.DMA,
        ],
    )
    def k(idx_hbm, tab_hbm, out_hbm, idx_v, rows_v, tile_v, gsem, ssem):
        wid = lax.axis_index("s") * _NC + lax.axis_index("c")
        base = wid * c_per_w
        n_bt = n_chunks // H
        pltpu.sync_copy(idx_hbm.at[pl.ds(base * _CH, c_per_w * _CH)], idx_v)

        def start_gather(j, b):
            idx_chunk = idx_v.at[pl.ds(j * _CH, _CH)]
            pltpu.make_async_copy(tab_hbm.at[idx_chunk], rows_v.at[b],
                                  gsem).start()

        def wait_gather(b):
            pltpu.make_async_copy(tab_hbm.at[idx_v.at[pl.ds(0, _CH)]],
                                  rows_v.at[b], gsem).wait()

        def start_store(j, b):
            c = base + j
            h = c // n_bt
            bt = lax.rem(c, n_bt)
            for dt in range(dt_n):
                pltpu.make_async_copy(tile_v.at[b, pl.ds(dt * 8, 8)],
                                      out_hbm.at[h, dt, bt], ssem).start()

        def wait_store(b):
            for dt in range(dt_n):
                pltpu.make_async_copy(tile_v.at[b, pl.ds(dt * 8, 8)],
                                      out_hbm.at[0, 0, 0], ssem).wait()

        iota = jnp.arange(_L, dtype=jnp.int32)
        perms = [(iota + k) & 15 for k in range(_L)]
        rowraw = [iota + _L * g for g in range(bt_n)]
        rowflat = [(iota + _L * g) * D for g in range(bt_n)]

        def transpose(b):
            src = rows_v.at[b]
            dst = tile_v.at[b]

            def cbody(c, carry):
                d0v = jnp.full((_L,), c * _L, dtype=jnp.int32)
                for k in range(_L):
                    dv = perms[k] + d0v
                    for g in range(bt_n):
                        val = plsc.load_gather(src, [rowraw[g], dv])
                        plsc.store_scatter(dst, [dv, rowraw[g]], val)
                return carry

            lax.fori_loop(0, D // _L, cbody, 0)

        for b in range(_NB):
            start_gather(b, b)

        def body(g, carry):
            for b in range(_NB):
                j = g * _NB + b
                wait_gather(b)

                @pl.when(g > 0)
                def _():
                    wait_store(b)

                transpose(b)
                start_store(j, b)

                @pl.when(g < n_groups - 1)
                def _():
                    start_gather(j + _NB, b)

            return carry

        lax.fori_loop(0, n_groups, body, 0)

        for b in range(_NB):
            wait_store(b)

    return k


def kernel(x, lookup_dict):
    B, H = x.shape
    V, D = lookup_dict.shape
    n = B * H
    n_chunks = n // _CH
    assert n % (_CH * _NW) == 0 and B % _CH == 0 and D % 8 == 0
    idx_flat = x.T.reshape(n).astype(jnp.int32)
    y = _lookup_call(n_chunks, V, H, D)(idx_flat, lookup_dict)
    return y.transpose(2, 4, 0, 1, 3).reshape(B, H, D)


# speedup vs baseline: 2.3552x; 1.0878x over previous
import functools

import jax
import jax.numpy as jnp
from jax import lax
from jax.experimental import pallas as pl
from jax.experimental.pallas import tpu as pltpu
from jax.experimental.pallas import tpu_sc as plsc

_NC = 2
_NS = 16
_NW = _NC * _NS
_CH = 128
_NB = 5
_L = 16


def _lookup_call(n_chunks, V, H, D):
    c_per_w = n_chunks // _NW
    n_groups = c_per_w // _NB
    bt_n = _CH // _L
    dt_n = D // 8
    mesh = plsc.VectorSubcoreMesh(core_axis_name="c", subcore_axis_name="s")

    @functools.partial(
        pl.kernel,
        mesh=mesh,
        compiler_params=pltpu.CompilerParams(use_tc_tiling_on_sc=False,
                                             needs_layout_passes=False),
        out_type=jax.ShapeDtypeStruct((H, dt_n, n_chunks // H, 8, _CH),
                                      jnp.float32),
        scratch_types=[
            pltpu.VMEM((c_per_w * _CH,), jnp.int32),
            pltpu.VMEM((_NB, _CH, D), jnp.float32),
            pltpu.VMEM((_NB, D, _CH), jnp.float32),
            pltpu.SemaphoreType.DMA,
            pltpu.SemaphoreType.DMA,
        ],
    )
    def k(idx_hbm, tab_hbm, out_hbm, idx_v, rows_v, tile_v, gsem, ssem):
        wid = lax.axis_index("s") * _NC + lax.axis_index("c")
        base = wid * c_per_w
        n_bt = n_chunks // H
        pltpu.sync_copy(idx_hbm.at[pl.ds(base * _CH, c_per_w * _CH)], idx_v)

        def start_gather(j, b):
            idx_chunk = idx_v.at[pl.ds(j * _CH, _CH)]
            pltpu.make_async_copy(tab_hbm.at[idx_chunk], rows_v.at[b],
                                  gsem).start()

        def wait_gather(b):
            pltpu.make_async_copy(tab_hbm.at[idx_v.at[pl.ds(0, _CH)]],
                                  rows_v.at[b], gsem).wait()

        def start_store(j, b):
            c = base + j
            h = c // n_bt
            bt = lax.rem(c, n_bt)
            for dt in range(dt_n):
                pltpu.make_async_copy(tile_v.at[b, pl.ds(dt * 8, 8)],
                                      out_hbm.at[h, dt, bt], ssem).start()

        def wait_store(b):
            for dt in range(dt_n):
                pltpu.make_async_copy(tile_v.at[b, pl.ds(dt * 8, 8)],
                                      out_hbm.at[0, 0, 0], ssem).wait()

        iota = jnp.arange(_L, dtype=jnp.int32)
        perms = [(iota + k) & 15 for k in range(_L)]
        rowraw = [iota + _L * g for g in range(bt_n)]
        rowflat = [(iota + _L * g) * D for g in range(bt_n)]

        def transpose(b):
            src = rows_v.at[b]
            dst = tile_v.at[b]

            @plsc.parallel_loop(0, D // _L, 1, unroll=2)
            def cbody(c):
                d0v = jnp.full((_L,), c * _L, dtype=jnp.int32)
                for k in range(_L):
                    dv = perms[k] + d0v
                    for g in range(bt_n):
                        val = plsc.load_gather(src, [rowraw[g], dv])
                        plsc.store_scatter(dst, [dv, rowraw[g]], val)

        for b in range(_NB):
            start_gather(b, b)

        def body(g, carry):
            for b in range(_NB):
                j = g * _NB + b
                wait_gather(b)

                @pl.when(g > 0)
                def _():
                    wait_store(b)

                transpose(b)
                start_store(j, b)

                @pl.when(g < n_groups - 1)
                def _():
                    start_gather(j + _NB, b)

            return carry

        lax.fori_loop(0, n_groups, body, 0)

        for b in range(_NB):
            wait_store(b)

    return k


def kernel(x, lookup_dict):
    B, H = x.shape
    V, D = lookup_dict.shape
    n = B * H
    n_chunks = n // _CH
    assert n % (_CH * _NW) == 0 and B % _CH == 0 and D % 8 == 0
    idx_flat = x.T.reshape(n).astype(jnp.int32)
    y = _lookup_call(n_chunks, V, H, D)(idx_flat, lookup_dict)
    return y.transpose(2, 4, 0, 1, 3).reshape(B, H, D)


# speedup vs baseline: 2.3652x; 1.0042x over previous
import functools

import jax
import jax.numpy as jnp
from jax import lax
from jax.experimental import pallas as pl
from jax.experimental.pallas import tpu as pltpu
from jax.experimental.pallas import tpu_sc as plsc

_NC = 2
_NS = 16
_NW = _NC * _NS
_CH = 128
_NB = 5
_L = 16


def _lookup_call(n_chunks, V, H, D):
    c_per_w = n_chunks // _NW
    n_groups = c_per_w // _NB
    bt_n = _CH // _L
    dt_n = D // 8
    mesh = plsc.VectorSubcoreMesh(core_axis_name="c", subcore_axis_name="s")

    @functools.partial(
        pl.kernel,
        mesh=mesh,
        compiler_params=pltpu.CompilerParams(use_tc_tiling_on_sc=False,
                                             needs_layout_passes=False),
        out_type=jax.ShapeDtypeStruct((H, dt_n, n_chunks // H, 8 * _CH),
                                      jnp.float32),
        scratch_types=[
            pltpu.VMEM((c_per_w * _CH,), jnp.int32),
            pltpu.VMEM((_NB, _CH, D), jnp.float32),
            pltpu.VMEM((_NB, D * _CH), jnp.float32),
            pltpu.SemaphoreType.DMA,
            pltpu.SemaphoreType.DMA,
        ],
    )
    def k(idx_hbm, tab_hbm, out_hbm, idx_v, rows_v, tile_v, gsem, ssem):
        wid = lax.axis_index("s") * _NC + lax.axis_index("c")
        base = wid * c_per_w
        n_bt = n_chunks // H
        pltpu.sync_copy(idx_hbm.at[pl.ds(base * _CH, c_per_w * _CH)], idx_v)

        def start_gather(j, b):
            idx_chunk = idx_v.at[pl.ds(j * _CH, _CH)]
            pltpu.make_async_copy(tab_hbm.at[idx_chunk], rows_v.at[b],
                                  gsem).start()

        def wait_gather(b):
            pltpu.make_async_copy(tab_hbm.at[idx_v.at[pl.ds(0, _CH)]],
                                  rows_v.at[b], gsem).wait()

        def start_store(j, b):
            c = base + j
            h = c // n_bt
            bt = lax.rem(c, n_bt)
            for dt in range(dt_n):
                pltpu.make_async_copy(tile_v.at[b, pl.ds(dt * 8 * _CH, 8 * _CH)],
                                      out_hbm.at[h, dt, bt], ssem).start()

        def wait_store(b):
            for dt in range(dt_n):
                pltpu.make_async_copy(tile_v.at[b, pl.ds(dt * 8 * _CH, 8 * _CH)],
                                      out_hbm.at[0, 0, 0], ssem).wait()

        iota = jnp.arange(_L, dtype=jnp.int32)
        perms = [(iota + k) & 15 for k in range(_L)]
        rowraw = [iota + _L * g for g in range(bt_n)]
        rowflat = [(iota + _L * g) * D for g in range(bt_n)]

        def transpose(b):
            src = rows_v.at[b]
            dst = tile_v.at[b]

            @plsc.parallel_loop(0, D // _L, 1, unroll=2)
            def cbody(c):
                d0v = jnp.full((_L,), c * _L, dtype=jnp.int32)
                for k in range(_L):
                    dv = perms[k] + d0v
                    sv = dv * _CH
                    for g in range(bt_n):
                        val = plsc.load_gather(src, [rowraw[g], dv])
                        plsc.store_scatter(dst, [sv + rowraw[g]], val)

        for b in range(_NB):
            start_gather(b, b)

        def body(g, carry):
            for b in range(_NB):
                j = g * _NB + b
                wait_gather(b)

                @pl.when(g > 0)
                def _():
                    wait_store(b)

                transpose(b)
                start_store(j, b)

                @pl.when(g < n_groups - 1)
                def _():
                    start_gather(j + _NB, b)

            return carry

        lax.fori_loop(0, n_groups, body, 0)

        for b in range(_NB):
            wait_store(b)

    return k


def kernel(x, lookup_dict):
    B, H = x.shape
    V, D = lookup_dict.shape
    n = B * H
    n_chunks = n // _CH
    assert n % (_CH * _NW) == 0 and B % _CH == 0 and D % 8 == 0
    idx_flat = x.T.reshape(n).astype(jnp.int32)
    y = _lookup_call(n_chunks, V, H, D)(idx_flat, lookup_dict)
    y = y.reshape(H, D // 8, B // _CH, 8, _CH)
    return y.transpose(2, 4, 0, 1, 3).reshape(B, H, D)


# speedup vs baseline: 3.3896x; 1.4331x over previous
import functools

import jax
import jax.numpy as jnp
from jax import lax
from jax.experimental import pallas as pl
from jax.experimental.pallas import tpu as pltpu
from jax.experimental.pallas import tpu_sc as plsc

_NC = 2
_NS = 16
_NW = _NC * _NS
_CH = 128
_NB = 5
_L = 16


def _lookup_call(n_chunks, V, H, D):
    c_per_w = n_chunks // _NW
    n_groups = c_per_w // _NB
    bt_n = _CH // _L
    dt_n = D // 8
    mesh = plsc.VectorSubcoreMesh(core_axis_name="c", subcore_axis_name="s")

    @functools.partial(
        pl.kernel,
        mesh=mesh,
        compiler_params=pltpu.CompilerParams(use_tc_tiling_on_sc=False,
                                             needs_layout_passes=False),
        out_type=jax.ShapeDtypeStruct((H, dt_n, n_chunks // H, 8, _CH),
                                      jnp.float32),
        scratch_types=[
            pltpu.VMEM((c_per_w * _CH,), jnp.int32),
            pltpu.VMEM((_NB, _CH, D), jnp.float32),
            pltpu.VMEM((_NB, D, _CH + 1), jnp.float32),
            pltpu.SemaphoreType.DMA,
            pltpu.SemaphoreType.DMA,
        ],
    )
    def k(idx_hbm, tab_hbm, out_hbm, idx_v, rows_v, tile_v, gsem, ssem):
        wid = lax.axis_index("s") * _NC + lax.axis_index("c")
        base = wid * c_per_w
        n_bt = n_chunks // H
        pltpu.sync_copy(idx_hbm.at[pl.ds(base * _CH, c_per_w * _CH)], idx_v)

        def start_gather(j, b):
            idx_chunk = idx_v.at[pl.ds(j * _CH, _CH)]
            pltpu.make_async_copy(tab_hbm.at[idx_chunk], rows_v.at[b],
                                  gsem).start()

        def wait_gather(b):
            pltpu.make_async_copy(tab_hbm.at[idx_v.at[pl.ds(0, _CH)]],
                                  rows_v.at[b], gsem).wait()

        def start_store(j, b):
            c = base + j
            h = c // n_bt
            bt = lax.rem(c, n_bt)
            for dt in range(dt_n):
                pltpu.make_async_copy(
                    tile_v.at[b, pl.ds(dt * 8, 8), pl.ds(0, _CH)],
                    out_hbm.at[h, dt, bt], ssem).start()

        def wait_store(b):
            for dt in range(dt_n):
                pltpu.make_async_copy(
                    tile_v.at[b, pl.ds(dt * 8, 8), pl.ds(0, _CH)],
                    out_hbm.at[0, 0, 0], ssem).wait()

        iota = jnp.arange(_L, dtype=jnp.int32)
        dvecs = [iota + c * _L for c in range(D // _L)]

        def transpose(b):
            src = rows_v.at[b]
            dst = tile_v.at[b]

            @plsc.parallel_loop(0, _CH, 1, unroll=4)
            def blbody(bl):
                blv = jnp.full((_L,), bl, dtype=jnp.int32)
                for c in range(D // _L):
                    val = src[bl, pl.ds(c * _L, _L)]
                    plsc.store_scatter(dst, [dvecs[c], blv], val)

        for b in range(_NB):
            start_gather(b, b)

        def body(g, carry):
            for b in range(_NB):
                j = g * _NB + b
                wait_gather(b)

                @pl.when(g > 0)
                def _():
                    wait_store(b)

                transpose(b)
                start_store(j, b)

                @pl.when(g < n_groups - 1)
                def _():
                    start_gather(j + _NB, b)

            return carry

        lax.fori_loop(0, n_groups, body, 0)

        for b in range(_NB):
            wait_store(b)

    return k


def kernel(x, lookup_dict):
    B, H = x.shape
    V, D = lookup_dict.shape
    n = B * H
    n_chunks = n // _CH
    assert n % (_CH * _NW) == 0 and B % _CH == 0 and D % 8 == 0
    idx_flat = x.T.reshape(n).astype(jnp.int32)
    y = _lookup_call(n_chunks, V, H, D)(idx_flat, lookup_dict)
    return y.transpose(2, 4, 0, 1, 3).reshape(B, H, D)
